# baseline (device time: 14470 ns/iter reference)
import jax
import jax.numpy as jnp
from jax import lax
from jax.experimental import pallas as pl
from jax.experimental.pallas import tpu as pltpu

N_DEV = 4
B = 2
SQ = 128
SKV = 128
H_LOC = 8
DH = 64
D = 512


def kernel(x, Wq, Wo, K_ext, V_ext):
    xf = x.reshape(B * SQ, D)
    kT = K_ext.transpose(0, 2, 3, 1).reshape(B, H_LOC * DH, SKV)
    vT = V_ext.transpose(0, 2, 3, 1).reshape(B, H_LOC * DH, SKV)

    def body(x_ref, wq_ref, wo_ref, kT_ref, vT_ref, out_ref,
             attnT_ref, comm_ref, send_sems, recv_sems):
        my = lax.axis_index("i")
        partner1 = my ^ 1
        partner2 = 3 - my

        barrier_sem = pltpu.get_barrier_semaphore()
        for nbr in (partner1, partner2):
            pl.semaphore_signal(
                barrier_sem, inc=1,
                device_id=(nbr,), device_id_type=pl.DeviceIdType.MESH,
            )
        pl.semaphore_wait(barrier_sem, 2)

        qT = lax.dot_general(
            wq_ref[...], x_ref[...], (((0,), (1,)), ((), ())),
            preferred_element_type=jnp.float32)

        for h in range(H_LOC):
            r = slice(h * DH, (h + 1) * DH)
            for b in range(B):
                qhb = qT[r, b * SQ:(b + 1) * SQ]
                khb = kT_ref[b, r, :]
                vhb = vT_ref[b, r, :]
                s = lax.dot_general(
                    qhb, khb, (((0,), (0,)), ((), ())),
                    preferred_element_type=jnp.float32) * 0.125
                m = jnp.max(s, axis=1, keepdims=True)
                p = jnp.exp(s - m)
                l = jnp.sum(p, axis=1, keepdims=True)
                pn = p / l
                oT = lax.dot_general(
                    vhb, pn, (((1,), (1,)), ((), ())),
                    preferred_element_type=jnp.float32)
                attnT_ref[r, b * SQ:(b + 1) * SQ] = oT

        out_ref[...] = lax.dot_general(
            attnT_ref[...], wo_ref[...], (((0,), (0,)), ((), ())),
            preferred_element_type=jnp.float32)

        for rnd, partner in enumerate((partner1, partner2)):
            rdma = pltpu.make_async_remote_copy(
                src_ref=out_ref,
                dst_ref=comm_ref.at[rnd],
                send_sem=send_sems.at[rnd],
                recv_sem=recv_sems.at[rnd],
                device_id=(partner,),
                device_id_type=pl.DeviceIdType.MESH,
            )
            rdma.start()
            rdma.wait()
            out_ref[...] += comm_ref[rnd]

    out = pl.pallas_call(
        body,
        out_shape=jax.ShapeDtypeStruct((B * SQ, D), jnp.float32),
        in_specs=[pl.BlockSpec(memory_space=pltpu.VMEM)] * 5,
        out_specs=pl.BlockSpec(memory_space=pltpu.VMEM),
        scratch_shapes=[
            pltpu.VMEM((D, B * SQ), jnp.float32),
            pltpu.VMEM((2, B * SQ, D), jnp.float32),
            pltpu.SemaphoreType.DMA((2,)),
            pltpu.SemaphoreType.DMA((2,)),
        ],
        compiler_params=pltpu.CompilerParams(collective_id=0),
    )(xf, Wq, Wo, kT, vT)
    return out.reshape(B, SQ, D)


# device time: 9565 ns/iter; 1.5128x vs baseline; 1.5128x over previous
import jax
import jax.numpy as jnp
from jax import lax
from jax.experimental import pallas as pl
from jax.experimental.pallas import tpu as pltpu

N_DEV = 4
B = 2
SQ = 128
SKV = 128
H_LOC = 8
DH = 64
D = 512


def kernel(x, Wq, Wo, K_ext, V_ext):
    xf = x.reshape(B * SQ, D)
    kf = K_ext.reshape(B, SKV, H_LOC * DH)
    vf = V_ext.reshape(B, SKV, H_LOC * DH)

    def body(x_ref, wq_ref, wo_ref, k_ref, v_ref, out_ref,
             attn_ref, send_ref, comm_ref, send_sems, recv_sems):
        my = lax.axis_index("i")
        partner1 = my ^ 1
        partner2 = 3 - my

        barrier_sem = pltpu.get_barrier_semaphore()
        for nbr in (partner1, partner2):
            pl.semaphore_signal(
                barrier_sem, inc=1,
                device_id=(nbr,), device_id_type=pl.DeviceIdType.MESH,
            )
        pl.semaphore_wait(barrier_sem, 2)

        bf16 = jnp.bfloat16
        q_all = jnp.dot(x_ref[...].astype(bf16), wq_ref[...].astype(bf16),
                        preferred_element_type=jnp.float32)
        q3 = q_all.astype(bf16).reshape(B, SQ, D)
        kbf = k_ref[...].astype(bf16)
        vbf = v_ref[...].astype(bf16)
        for h in range(H_LOC):
            c = slice(h * DH, (h + 1) * DH)
            qh = q3[:, :, c]
            s = lax.dot_general(
                qh, kbf[:, :, c], (((2,), (2,)), ((0,), (0,))),
                preferred_element_type=jnp.float32) * 0.125
            p = jnp.exp(s)
            l = jnp.sum(p, axis=2, keepdims=True)
            o = lax.dot_general(
                p.astype(bf16), vbf[:, :, c], (((2,), (1,)), ((0,), (0,))),
                preferred_element_type=jnp.float32) / l
            attn_ref[:, c] = o.astype(bf16).reshape(B * SQ, DH)

        out_ref[...] = jnp.dot(attn_ref[...], wo_ref[...].astype(bf16),
                               preferred_element_type=jnp.float32)

        for rnd, partner in enumerate((partner1, partner2)):
            send_ref[...] = out_ref[...].astype(bf16)
            rdma = pltpu.make_async_remote_copy(
                src_ref=send_ref,
                dst_ref=comm_ref.at[rnd],
                send_sem=send_sems.at[rnd],
                recv_sem=recv_sems.at[rnd],
                device_id=(partner,),
                device_id_type=pl.DeviceIdType.MESH,
            )
            rdma.start()
            rdma.wait()
            out_ref[...] += comm_ref[rnd].astype(jnp.float32)

    out = pl.pallas_call(
        body,
        out_shape=jax.ShapeDtypeStruct((B * SQ, D), jnp.float32),
        in_specs=[pl.BlockSpec(memory_space=pltpu.VMEM)] * 5,
        out_specs=pl.BlockSpec(memory_space=pltpu.VMEM),
        scratch_shapes=[
            pltpu.VMEM((B * SQ, D), jnp.bfloat16),
            pltpu.VMEM((B * SQ, D), jnp.bfloat16),
            pltpu.VMEM((2, B * SQ, D), jnp.bfloat16),
            pltpu.SemaphoreType.DMA((2,)),
            pltpu.SemaphoreType.DMA((2,)),
        ],
        compiler_params=pltpu.CompilerParams(collective_id=0),
    )(xf, Wq, Wo, kf, vf)
    return out.reshape(B, SQ, D)
